# baseline (speedup 1.0000x reference)
import functools

import jax
import jax.numpy as jnp
from jax import lax
from jax.experimental import pallas as pl
from jax.experimental.pallas import tpu as pltpu
from jax.experimental.pallas import tpu_sc as plsc

_D = 32
_SR = 128
_HP = 64


def _sc_transe(lhs, rel, rhs, ent2, rel2, B):
    info = plsc.get_sparse_core_info()
    nw = info.num_cores * info.num_subcores
    bpw = B // nw
    npass = bpw // _HP

    mesh = plsc.VectorSubcoreMesh(core_axis_name="c", subcore_axis_name="s")

    @functools.partial(
        pl.kernel,
        mesh=mesh,
        compiler_params=pltpu.CompilerParams(
            needs_layout_passes=False, skip_device_barrier=True),
        out_type=jax.ShapeDtypeStruct((B,), jnp.float32),
        scratch_types=[
            pltpu.VMEM((_HP,), jnp.int32),
            pltpu.VMEM((_HP,), jnp.int32),
            pltpu.VMEM((_HP,), jnp.int32),
            pltpu.VMEM((_HP,), jnp.int32),
            pltpu.VMEM((_HP,), jnp.int32),
            pltpu.VMEM((_HP,), jnp.int32),
            pltpu.VMEM((_HP, _SR), jnp.float32),
            pltpu.VMEM((_HP, _SR), jnp.float32),
            pltpu.VMEM((_HP, _SR), jnp.float32),
            pltpu.VMEM((bpw,), jnp.float32),
            pltpu.SemaphoreType.DMA,
            pltpu.SemaphoreType.DMA,
        ],
    )
    def transe_kernel(lhs_hbm, rel_hbm, rhs_hbm, ent_hbm, relm_hbm, out_hbm,
                      li, ri, hi, l4, r4, h4, lv, rv, hv, ov, sem_g, sem_o):
        wid = lax.axis_index("s") * info.num_cores + lax.axis_index("c")
        b0 = wid * bpw
        lanes = lax.iota(jnp.int32, 16)

        def rsqrt(n):
            y = lax.bitcast_convert_type(n, jnp.int32)
            x = lax.bitcast_convert_type(0x5F3759DF - (y >> 1), jnp.float32)
            for _ in range(3):
                x = x * (1.5 - 0.5 * n * x * x)
            return x

        def run_pass(p, _):
            o = b0 + p * _HP
            pltpu.sync_copy(lhs_hbm.at[pl.ds(o, _HP)], li)
            pltpu.sync_copy(rel_hbm.at[pl.ds(o, _HP)], ri)
            pltpu.sync_copy(rhs_hbm.at[pl.ds(o, _HP)], hi)
            for k in range(_HP // 16):
                sl16 = pl.ds(k * 16, 16)
                l4[sl16] = li[sl16] >> 2
                r4[sl16] = ri[sl16] >> 2
                h4[sl16] = hi[sl16] >> 2
            cl = pltpu.async_copy(ent_hbm.at[l4], lv, sem_g)
            cr = pltpu.async_copy(relm_hbm.at[r4], rv, sem_g)
            ch = pltpu.async_copy(ent_hbm.at[h4], hv, sem_g)
            cl.wait()
            cr.wait()
            ch.wait()

            def group(g, _):
                svec = g * 16 + lanes
                sl16 = pl.ds(g * 16, 16)
                cbl = (li[sl16] & 3) * _D
                cbr = (ri[sl16] & 3) * _D
                cbh = (hi[sl16] & 3) * _D
                nll = jnp.zeros((16,), jnp.float32)
                nrr = jnp.zeros((16,), jnp.float32)
                nhh = jnp.zeros((16,), jnp.float32)
                nlr = jnp.zeros((16,), jnp.float32)
                nlh = jnp.zeros((16,), jnp.float32)
                nrh = jnp.zeros((16,), jnp.float32)
                for j in range(_D):
                    xl = plsc.load_gather(lv, [svec, cbl + j])
                    xr = plsc.load_gather(rv, [svec, cbr + j])
                    xh = plsc.load_gather(hv, [svec, cbh + j])
                    nll = nll + xl * xl
                    nrr = nrr + xr * xr
                    nhh = nhh + xh * xh
                    nlr = nlr + xl * xr
                    nlh = nlh + xl * xh
                    nrh = nrh + xr * xh
                sl = jnp.minimum(1.0, rsqrt(nll))
                sr = jnp.minimum(1.0, rsqrt(nrr))
                sh = jnp.minimum(1.0, rsqrt(nhh))
                e2 = jnp.maximum(
                    sl * sl * nll + sr * sr * nrr + sh * sh * nhh
                    + 2.0 * (sl * sr * nlr - sl * sh * nlh - sr * sh * nrh),
                    0.0)
                ov[pl.ds(p * _HP + g * 16, 16)] = e2 * rsqrt(
                    jnp.maximum(e2, 1e-30))
                return 0

            lax.fori_loop(0, _HP // 16, group, 0)
            return 0

        lax.fori_loop(0, npass, run_pass, 0)
        pltpu.async_copy(ov, out_hbm.at[pl.ds(b0, bpw)], sem_o)
        pltpu.make_async_copy(ov, out_hbm.at[pl.ds(b0, bpw)], sem_o).wait()

    return transe_kernel(lhs, rel, rhs, ent2, rel2)


def kernel(triplets, ent_embeds, rel_embeds):
    B = triplets.shape[0]
    lhs = triplets[:, 0]
    rel = triplets[:, 1]
    rhs = triplets[:, 2]
    ent2 = ent_embeds.reshape(-1, _SR)
    rel2 = rel_embeds.reshape(-1, _SR)
    return _sc_transe(lhs, rel, rhs, ent2, rel2, B)


# speedup vs baseline: 1.6108x; 1.6108x over previous
import functools

import jax
import jax.numpy as jnp
from jax import lax
from jax.experimental import pallas as pl
from jax.experimental.pallas import tpu as pltpu
from jax.experimental.pallas import tpu_sc as plsc

_D = 32
_CH = 32
_HP = 256


def _sc_transe(lhs, rel, rhs, ent_embeds, rel_embeds, B):
    D = _D
    info = plsc.get_sparse_core_info()
    nw = info.num_cores * info.num_subcores
    bpw = B // nw
    npass = bpw // _HP
    nchp = _HP // _CH

    mesh = plsc.VectorSubcoreMesh(core_axis_name="c", subcore_axis_name="s")

    @functools.partial(
        pl.kernel,
        mesh=mesh,
        compiler_params=pltpu.CompilerParams(
            needs_layout_passes=False, skip_device_barrier=True),
        out_type=jax.ShapeDtypeStruct((B,), jnp.float32),
        scratch_types=[
            pltpu.VMEM((bpw,), jnp.int32),
            pltpu.VMEM((bpw,), jnp.int32),
            pltpu.VMEM((bpw,), jnp.int32),
            pltpu.VMEM((_HP, D), jnp.float32),
            pltpu.VMEM((_HP, D), jnp.float32),
            pltpu.VMEM((_HP, D), jnp.float32),
            pltpu.VMEM((bpw,), jnp.float32),
            pltpu.SemaphoreType.DMA,
            pltpu.SemaphoreType.DMA,
        ],
    )
    def transe_kernel(lhs_hbm, rel_hbm, rhs_hbm, ent_hbm, relm_hbm, out_hbm,
                      li, ri, hi, lv, rv, hv, ov, sem_g, sem_o):
        wid = lax.axis_index("s") * info.num_cores + lax.axis_index("c")
        b0 = wid * bpw
        pltpu.sync_copy(lhs_hbm.at[pl.ds(b0, bpw)], li)
        pltpu.sync_copy(rel_hbm.at[pl.ds(b0, bpw)], ri)
        pltpu.sync_copy(rhs_hbm.at[pl.ds(b0, bpw)], hi)
        lanes = lax.iota(jnp.int32, 16)

        def fire(p, c):
            for half in range(_CH // 16):
                base = p * _HP + c * _CH + half * 16
                v1 = li[pl.ds(base, 16)]
                v2 = ri[pl.ds(base, 16)]
                v3 = hi[pl.ds(base, 16)]
                for i in range(16):
                    s = c * _CH + half * 16 + i
                    pltpu.async_copy(ent_hbm.at[v1[i]], lv.at[s], sem_g)
                    pltpu.async_copy(relm_hbm.at[v2[i]], rv.at[s], sem_g)
                    pltpu.async_copy(ent_hbm.at[v3[i]], hv.at[s], sem_g)

        def drain_chunk():
            pltpu.make_async_copy(
                ent_hbm.at[pl.ds(0, 3 * _CH)], lv.at[pl.ds(0, 3 * _CH)],
                sem_g).wait()

        def rsqrt(n):
            y = lax.bitcast_convert_type(n, jnp.int32)
            x = lax.bitcast_convert_type(0x5F3759DF - (y >> 1), jnp.float32)
            for _ in range(3):
                x = x * (1.5 - 0.5 * n * x * x)
            return x

        def group(p, g):
            svec = g * 16 + lanes
            nll = jnp.zeros((16,), jnp.float32)
            nrr = jnp.zeros((16,), jnp.float32)
            nhh = jnp.zeros((16,), jnp.float32)
            nlr = jnp.zeros((16,), jnp.float32)
            nlh = jnp.zeros((16,), jnp.float32)
            nrh = jnp.zeros((16,), jnp.float32)
            for j in range(_D):
                jv = jnp.full((16,), j, jnp.int32)
                xl = plsc.load_gather(lv, [svec, jv])
                xr = plsc.load_gather(rv, [svec, jv])
                xh = plsc.load_gather(hv, [svec, jv])
                nll = nll + xl * xl
                nrr = nrr + xr * xr
                nhh = nhh + xh * xh
                nlr = nlr + xl * xr
                nlh = nlh + xl * xh
                nrh = nrh + xr * xh
            sl = jnp.minimum(1.0, rsqrt(nll))
            sr = jnp.minimum(1.0, rsqrt(nrr))
            sh = jnp.minimum(1.0, rsqrt(nhh))
            e2 = jnp.maximum(
                sl * sl * nll + sr * sr * nrr + sh * sh * nhh
                + 2.0 * (sl * sr * nlr - sl * sh * nlh - sr * sh * nrh),
                0.0)
            ov[pl.ds(p * _HP + g * 16, 16)] = e2 * rsqrt(
                jnp.maximum(e2, 1e-30))

        def run_pass(p, _):
            def fire_drain(c, _):
                fire(p, c)
                drain_chunk()
                return 0

            fire(p, 0)
            lax.fori_loop(1, nchp, fire_drain, 0)
            drain_chunk()

            def comp(g, _):
                group(p, g)
                return 0

            lax.fori_loop(0, _HP // 16, comp, 0)
            return 0

        lax.fori_loop(0, npass, run_pass, 0)
        pltpu.async_copy(ov, out_hbm.at[pl.ds(b0, bpw)], sem_o)
        pltpu.make_async_copy(ov, out_hbm.at[pl.ds(b0, bpw)], sem_o).wait()

    return transe_kernel(lhs, rel, rhs, ent_embeds, rel_embeds)


def kernel(triplets, ent_embeds, rel_embeds):
    B = triplets.shape[0]
    lhs = triplets[:, 0]
    rel = triplets[:, 1]
    rhs = triplets[:, 2]
    return _sc_transe(lhs, rel, rhs, ent_embeds, rel_embeds, B)
